# baseline (speedup 1.0000x reference)
import functools

import jax
import jax.numpy as jnp
from jax.experimental import pallas as pl

_K = 16
_ALPHA = 1.05
_ROWS = 1024
_INF = 3.0e38


def _knn_block_kernel(pts_ref, knn_ref):
    i = pl.program_id(1)
    n = pts_ref.shape[2]
    r0 = i * _ROWS

    dist2 = jnp.zeros((_ROWS, n), dtype=jnp.float32)
    for d in range(3):
        col = pts_ref[0, d, :].reshape(1, n)
        row = pts_ref[0, d, pl.ds(r0, _ROWS)].reshape(_ROWS, 1)
        diff = row - col
        dist2 = dist2 + diff * diff
    dist2 = jnp.maximum(dist2, 1e-12)

    colbits = jax.lax.broadcasted_iota(jnp.int32, (1, n), 1)
    bits = jax.lax.bitcast_convert_type(dist2, jnp.int32)
    bits = jax.lax.bitwise_or(jax.lax.bitwise_and(bits, ~jnp.int32(2047)),
                              colbits)
    keyed = jax.lax.bitcast_convert_type(bits, jnp.float32)

    rowids = r0 + jax.lax.broadcasted_iota(jnp.int32, (_ROWS, 1), 0)
    keyed = jnp.where(colbits == rowids, _INF, keyed)

    total = jnp.zeros((_ROWS, 1), dtype=jnp.float32)
    for _ in range(_K):
        m = jnp.min(keyed, axis=1, keepdims=True)
        total = total + jnp.sqrt(m)
        keyed = jnp.where(keyed == m, _INF, keyed)

    knn = total * (1.0 / _K)
    knn_ref[0, 0, :, :] = knn.reshape(1, _ROWS)


def _loss_kernel(knn_ref, out_ref, *, n):
    x = knn_ref[...]
    mean = jnp.mean(x, axis=1, keepdims=True)
    c = x - mean
    var = jnp.sum(c * c, axis=1, keepdims=True) * (1.0 / (n - 1))
    thr = mean + _ALPHA * jnp.sqrt(var)
    pen = jnp.where(x > thr, x, jnp.zeros_like(x))
    out_ref[:, :] = jnp.mean(pen).reshape(1, 1)


def kernel(pcs):
    b, n, _ = pcs.shape
    pts = jnp.transpose(pcs, (0, 2, 1))

    nblk = n // _ROWS
    knn = pl.pallas_call(
        _knn_block_kernel,
        grid=(b, nblk),
        in_specs=[pl.BlockSpec((1, 3, n), lambda bb, ii: (bb, 0, 0))],
        out_specs=pl.BlockSpec((1, 1, 1, _ROWS), lambda bb, ii: (bb, ii, 0, 0)),
        out_shape=jax.ShapeDtypeStruct((b, nblk, 1, _ROWS), jnp.float32),
    )(pts)
    knn = knn.reshape(b, n)

    loss = pl.pallas_call(
        functools.partial(_loss_kernel, n=n),
        in_specs=[pl.BlockSpec((b, n), lambda: (0, 0))],
        out_specs=pl.BlockSpec((1, 1), lambda: (0, 0)),
        out_shape=jax.ShapeDtypeStruct((1, 1), jnp.float32),
    )(knn)
    return loss.reshape(())


# speedup vs baseline: 1.0008x; 1.0008x over previous
import functools

import jax
import jax.numpy as jnp
from jax.experimental import pallas as pl

_K = 16
_ALPHA = 1.05
_ROWS = 512
_INF = 3.0e38


def _knn_block_kernel(pts_ref, knn_ref):
    i = pl.program_id(1)
    n = pts_ref.shape[2]
    r0 = i * _ROWS

    dist2 = jnp.zeros((_ROWS, n), dtype=jnp.float32)
    for d in range(3):
        col = pts_ref[0, d, :].reshape(1, n)
        row = pts_ref[0, d, pl.ds(r0, _ROWS)].reshape(_ROWS, 1)
        diff = row - col
        dist2 = dist2 + diff * diff
    dist2 = jnp.maximum(dist2, 1e-12)

    colbits = jax.lax.broadcasted_iota(jnp.int32, (1, n), 1)
    bits = jax.lax.bitcast_convert_type(dist2, jnp.int32)
    bits = jax.lax.bitwise_or(jax.lax.bitwise_and(bits, ~jnp.int32(2047)),
                              colbits)
    keyed = jax.lax.bitcast_convert_type(bits, jnp.float32)

    rowids = r0 + jax.lax.broadcasted_iota(jnp.int32, (_ROWS, 1), 0)
    keyed = jnp.where(colbits == rowids, _INF, keyed)

    total = jnp.zeros((_ROWS, 1), dtype=jnp.float32)
    for _ in range(_K):
        m = jnp.min(keyed, axis=1, keepdims=True)
        total = total + jnp.sqrt(m)
        keyed = jnp.where(keyed == m, _INF, keyed)

    knn = total * (1.0 / _K)
    knn_ref[0, 0, :, :] = knn.reshape(1, _ROWS)


def _loss_kernel(knn_ref, out_ref, *, n):
    x = knn_ref[...]
    mean = jnp.mean(x, axis=1, keepdims=True)
    c = x - mean
    var = jnp.sum(c * c, axis=1, keepdims=True) * (1.0 / (n - 1))
    thr = mean + _ALPHA * jnp.sqrt(var)
    pen = jnp.where(x > thr, x, jnp.zeros_like(x))
    out_ref[:, :] = jnp.mean(pen).reshape(1, 1)


def kernel(pcs):
    b, n, _ = pcs.shape
    pts = jnp.transpose(pcs, (0, 2, 1))

    nblk = n // _ROWS
    knn = pl.pallas_call(
        _knn_block_kernel,
        grid=(b, nblk),
        in_specs=[pl.BlockSpec((1, 3, n), lambda bb, ii: (bb, 0, 0))],
        out_specs=pl.BlockSpec((1, 1, 1, _ROWS), lambda bb, ii: (bb, ii, 0, 0)),
        out_shape=jax.ShapeDtypeStruct((b, nblk, 1, _ROWS), jnp.float32),
    )(pts)
    knn = knn.reshape(b, n)

    loss = pl.pallas_call(
        functools.partial(_loss_kernel, n=n),
        in_specs=[pl.BlockSpec((b, n), lambda: (0, 0))],
        out_specs=pl.BlockSpec((1, 1), lambda: (0, 0)),
        out_shape=jax.ShapeDtypeStruct((1, 1), jnp.float32),
    )(knn)
    return loss.reshape(())


# speedup vs baseline: 1.0010x; 1.0002x over previous
import functools

import jax
import jax.numpy as jnp
from jax.experimental import pallas as pl

_K = 16
_ALPHA = 1.05
_ROWS = 512
_INF = 3.0e38


def _knn_block_kernel(pts_ref, knn_ref):
    i = pl.program_id(1)
    n = pts_ref.shape[2]
    r0 = i * _ROWS

    dist2 = jnp.zeros((_ROWS, n), dtype=jnp.float32)
    for d in range(3):
        col = pts_ref[0, d, :].reshape(1, n)
        row = pts_ref[0, d, pl.ds(r0, _ROWS)].reshape(_ROWS, 1)
        diff = row - col
        dist2 = dist2 + diff * diff
    dist2 = jnp.maximum(dist2, 1e-12)

    colbits = jax.lax.broadcasted_iota(jnp.int32, (1, n), 1)
    bits = jax.lax.bitcast_convert_type(dist2, jnp.int32)
    bits = jax.lax.bitwise_or(jax.lax.bitwise_and(bits, ~jnp.int32(2047)),
                              colbits)
    keyed = jax.lax.bitcast_convert_type(bits, jnp.float32)

    rowids = r0 + jax.lax.broadcasted_iota(jnp.int32, (_ROWS, 1), 0)
    keyed = jnp.where(colbits == rowids, _INF, keyed)

    total = jnp.zeros((_ROWS, 1), dtype=jnp.float32)
    for t in range(_K):
        m = jnp.min(keyed, axis=1, keepdims=True)
        total = total + jnp.sqrt(m)
        if t < _K - 1:
            keyed = jnp.where(keyed == m, _INF, keyed)

    knn = total * (1.0 / _K)
    knn_ref[0, 0, :, :] = knn.reshape(1, _ROWS)


def _loss_kernel(knn_ref, out_ref, *, n):
    x = knn_ref[...]
    mean = jnp.mean(x, axis=1, keepdims=True)
    c = x - mean
    var = jnp.sum(c * c, axis=1, keepdims=True) * (1.0 / (n - 1))
    thr = mean + _ALPHA * jnp.sqrt(var)
    pen = jnp.where(x > thr, x, jnp.zeros_like(x))
    out_ref[:, :] = jnp.mean(pen).reshape(1, 1)


def kernel(pcs):
    b, n, _ = pcs.shape
    pts = jnp.transpose(pcs, (0, 2, 1))

    nblk = n // _ROWS
    knn = pl.pallas_call(
        _knn_block_kernel,
        grid=(b, nblk),
        in_specs=[pl.BlockSpec((1, 3, n), lambda bb, ii: (bb, 0, 0))],
        out_specs=pl.BlockSpec((1, 1, 1, _ROWS), lambda bb, ii: (bb, ii, 0, 0)),
        out_shape=jax.ShapeDtypeStruct((b, nblk, 1, _ROWS), jnp.float32),
    )(pts)
    knn = knn.reshape(b, n)

    loss = pl.pallas_call(
        functools.partial(_loss_kernel, n=n),
        in_specs=[pl.BlockSpec((b, n), lambda: (0, 0))],
        out_specs=pl.BlockSpec((1, 1), lambda: (0, 0)),
        out_shape=jax.ShapeDtypeStruct((1, 1), jnp.float32),
    )(knn)
    return loss.reshape(())
